# baseline (speedup 1.0000x reference)
import functools

import jax
import jax.numpy as jnp
from jax import lax
from jax.experimental import pallas as pl
from jax.experimental.pallas import tpu as pltpu
from jax.experimental.pallas import tpu_sc as plsc

BM = 512
NUM_WORKERS = 32


def _routing_metadata(expert_indices, num_experts):
    T, K = expert_indices.shape
    N = T * K
    E = num_experts
    n_blocks = N // BM + E - 1

    flat_e = expert_indices.reshape(-1).astype(jnp.int32)
    onehot = flat_e[None, :] == jnp.arange(E, dtype=jnp.int32)[:, None]
    incl = jnp.cumsum(onehot.astype(jnp.int32), axis=1)
    counts = incl[:, -1]
    padded_counts = ((counts + BM - 1) // BM) * BM
    gstart = jnp.concatenate(
        [jnp.zeros((1,), jnp.int32),
         jnp.cumsum(padded_counts)[:-1].astype(jnp.int32)])
    pos = jnp.sum(
        jnp.where(onehot, incl + (gstart - 1)[:, None], 0), axis=0)

    pos_k = pos.reshape(T, K).T
    pos_kt = pos_k.reshape(-1)

    bstart = jnp.arange(n_blocks, dtype=jnp.int32) * BM
    gend = gstart + padded_counts
    block_expert = jnp.sum(
        (bstart[:, None] >= gend[None, :]).astype(jnp.int32), axis=1)
    block_expert = jnp.minimum(block_expert, E - 1).astype(jnp.int32)
    return block_expert, pos_k, pos_kt, n_blocks


def _sc_permute(x, pos_k, npad, chunk):
    T, H = x.shape
    K = pos_k.shape[0]
    t_per_w = T // NUM_WORKERS
    nchunks = t_per_w // chunk
    idx3 = pos_k.reshape(K, NUM_WORKERS, nchunks, chunk)
    mesh = plsc.VectorSubcoreMesh(core_axis_name="c", subcore_axis_name="s")

    @functools.partial(
        pl.kernel,
        mesh=mesh,
        out_type=jax.ShapeDtypeStruct((npad, H), x.dtype),
        scratch_types=[
            pltpu.VMEM((K, nchunks, chunk), jnp.int32),
            pltpu.VMEM((chunk, H), x.dtype),
            pltpu.SemaphoreType.DMA,
        ],
    )
    def k(x_hbm, idx_hbm, out_hbm, idx_v, rows_v, sem):
        wid = lax.axis_index("s") * 2 + lax.axis_index("c")
        pltpu.async_copy(idx_hbm.at[:, wid], idx_v, sem).wait()
        base = wid * t_per_w

        @pl.loop(0, nchunks)
        def _(c):
            pltpu.async_copy(x_hbm.at[pl.ds(base + c * chunk, chunk)], rows_v,
                             sem).wait()
            for kk in range(K):
                pltpu.async_copy(rows_v, out_hbm.at[idx_v.at[kk].at[c]],
                                 sem).wait()

    return k(x, idx3)


def _sc_unpermute(y, pos_kt, chunk):
    H = y.shape[1]
    n = pos_kt.shape[0]
    rows_per_w = n // NUM_WORKERS
    nchunks = rows_per_w // chunk
    idx3 = pos_kt.reshape(NUM_WORKERS, nchunks, chunk)
    mesh = plsc.VectorSubcoreMesh(core_axis_name="c", subcore_axis_name="s")

    @functools.partial(
        pl.kernel,
        mesh=mesh,
        out_type=jax.ShapeDtypeStruct((n, H), y.dtype),
        scratch_types=[
            pltpu.VMEM((nchunks, chunk), jnp.int32),
            pltpu.VMEM((chunk, H), y.dtype),
            pltpu.SemaphoreType.DMA,
        ],
    )
    def k(y_hbm, idx_hbm, out_hbm, idx_v, rows_v, sem):
        wid = lax.axis_index("s") * 2 + lax.axis_index("c")
        pltpu.async_copy(idx_hbm.at[wid], idx_v, sem).wait()
        base = wid * rows_per_w

        @pl.loop(0, nchunks)
        def _(c):
            pltpu.async_copy(y_hbm.at[idx_v.at[c]], rows_v, sem).wait()
            pltpu.async_copy(rows_v, out_hbm.at[pl.ds(base + c * chunk, chunk)],
                             sem).wait()

    return k(y, idx3)


def _cast_body(w_ref, o_ref):
    o_ref[...] = w_ref[...].astype(jnp.bfloat16)


def _cast_bf16(w, interpret=False):
    E = w.shape[0]
    spec = pl.BlockSpec((1,) + w.shape[1:], lambda i: (i, 0, 0))
    return pl.pallas_call(
        _cast_body,
        grid=(E,),
        in_specs=[spec],
        out_specs=spec,
        out_shape=jax.ShapeDtypeStruct(w.shape, jnp.bfloat16),
        interpret=interpret,
    )(w)


def _ffn_body(be_ref, xs_ref, w1_ref, w3_ref, w2_ref, y_ref):
    xb = xs_ref[...].astype(jnp.bfloat16)
    g = jnp.dot(xb, w1_ref[0], preferred_element_type=jnp.float32)
    u = jnp.dot(xb, w3_ref[0], preferred_element_type=jnp.float32)
    h = ((g * lax.logistic(g)) * u).astype(jnp.bfloat16)
    y_ref[...] = jnp.dot(h, w2_ref[0], preferred_element_type=jnp.float32)


def _grouped_ffn(block_expert, xs, w1, w3, w2, n_blocks, interpret=False):
    npad, H = xs.shape
    I = w1.shape[2]
    grid_spec = pltpu.PrefetchScalarGridSpec(
        num_scalar_prefetch=1,
        grid=(n_blocks,),
        in_specs=[
            pl.BlockSpec((BM, H), lambda i, be: (i, 0)),
            pl.BlockSpec((1, H, I), lambda i, be: (be[i], 0, 0)),
            pl.BlockSpec((1, H, I), lambda i, be: (be[i], 0, 0)),
            pl.BlockSpec((1, I, H), lambda i, be: (be[i], 0, 0)),
        ],
        out_specs=pl.BlockSpec((BM, H), lambda i, be: (i, 0)),
    )
    return pl.pallas_call(
        _ffn_body,
        grid_spec=grid_spec,
        out_shape=jax.ShapeDtypeStruct((npad, H), jnp.float32),
        interpret=interpret,
    )(block_expert, xs, w1, w3, w2)


def _combine_body(*refs):
    out_ref = refs[-1]
    acc = refs[0][...] * refs[1][...]
    for j in range(2, len(refs) - 1, 2):
        acc = acc + refs[j][...] * refs[j + 1][...]
    out_ref[...] = acc


def _combine(yu, expert_weights, T, K, interpret=False):
    H = yu.shape[1]
    BT = 512
    nb = T // BT
    ew_col = expert_weights.T.reshape(K * T, 1)
    in_specs = []
    args = []
    for k in range(K):
        in_specs.append(
            pl.BlockSpec((BT, H),
                         functools.partial(lambda k, i: (k * nb + i, 0), k)))
        args.append(yu)
        in_specs.append(
            pl.BlockSpec((BT, 1),
                         functools.partial(lambda k, i: (k * nb + i, 0), k)))
        args.append(ew_col)
    return pl.pallas_call(
        _combine_body,
        grid=(nb,),
        in_specs=in_specs,
        out_specs=pl.BlockSpec((BT, H), lambda i: (i, 0)),
        out_shape=jax.ShapeDtypeStruct((T, H), jnp.float32),
        interpret=interpret,
    )(*args)


def kernel(x, expert_indices, expert_weights, w1, w2, w3):
    T, H = x.shape
    K = expert_indices.shape[1]
    E = w1.shape[0]

    block_expert, pos_k, pos_kt, n_blocks = _routing_metadata(
        expert_indices, E)
    npad = n_blocks * BM

    xs = _sc_permute(x, pos_k, npad, chunk=16)
    y = _grouped_ffn(block_expert, xs, _cast_bf16(w1), _cast_bf16(w3),
                     _cast_bf16(w2), n_blocks)
    yu = _sc_unpermute(y, pos_kt, chunk=32)
    return _combine(yu, expert_weights.astype(jnp.float32), T, K)


# speedup vs baseline: 1.3219x; 1.3219x over previous
import functools

import jax
import jax.numpy as jnp
from jax import lax
from jax.experimental import pallas as pl
from jax.experimental.pallas import tpu as pltpu
from jax.experimental.pallas import tpu_sc as plsc

BM = 512
NUM_WORKERS = 32


def _routing_metadata(expert_indices, num_experts):
    T, K = expert_indices.shape
    N = T * K
    E = num_experts
    n_blocks = N // BM + E - 1

    flat_e = expert_indices.reshape(-1).astype(jnp.int32)
    onehot = flat_e[None, :] == jnp.arange(E, dtype=jnp.int32)[:, None]
    incl = jnp.cumsum(onehot.astype(jnp.int32), axis=1)
    counts = incl[:, -1]
    padded_counts = ((counts + BM - 1) // BM) * BM
    gstart = jnp.concatenate(
        [jnp.zeros((1,), jnp.int32),
         jnp.cumsum(padded_counts)[:-1].astype(jnp.int32)])
    pos = jnp.sum(
        jnp.where(onehot, incl + (gstart - 1)[:, None], 0), axis=0)

    pos_k = pos.reshape(T, K).T
    pos_kt = pos_k.reshape(-1)

    bstart = jnp.arange(n_blocks, dtype=jnp.int32) * BM
    gend = gstart + padded_counts
    block_expert = jnp.sum(
        (bstart[:, None] >= gend[None, :]).astype(jnp.int32), axis=1)
    block_expert = jnp.minimum(block_expert, E - 1).astype(jnp.int32)
    block_valid = (bstart < gend[-1]).astype(jnp.int32)
    return block_expert, block_valid, pos_k, pos_kt, n_blocks


def _sc_permute(x, pos_k, npad, chunk):
    T, H = x.shape
    K = pos_k.shape[0]
    t_per_w = T // NUM_WORKERS
    nchunks = t_per_w // chunk
    idx3 = pos_k.reshape(K, NUM_WORKERS, nchunks, chunk)
    mesh = plsc.VectorSubcoreMesh(core_axis_name="c", subcore_axis_name="s")

    @functools.partial(
        pl.kernel,
        mesh=mesh,
        out_type=jax.ShapeDtypeStruct((npad, H), x.dtype),
        scratch_types=[
            pltpu.VMEM((K, nchunks, chunk), jnp.int32),
            pltpu.VMEM((chunk, H), x.dtype),
            pltpu.VMEM((chunk, H), x.dtype),
            pltpu.SemaphoreType.DMA,
            pltpu.SemaphoreType.DMA,
            pltpu.SemaphoreType.DMA,
        ],
    )
    def k(x_hbm, idx_hbm, out_hbm, idx_v, buf_a, buf_b, sem_a, sem_b, sem_s):
        wid = lax.axis_index("s") * 2 + lax.axis_index("c")
        pltpu.async_copy(idx_hbm.at[:, wid], idx_v, sem_a).wait()
        base = wid * t_per_w

        @pl.loop(0, nchunks, step=2)
        def _(c):
            ra = pltpu.async_copy(
                x_hbm.at[pl.ds(base + c * chunk, chunk)], buf_a, sem_a)
            rb = pltpu.async_copy(
                x_hbm.at[pl.ds(base + (c + 1) * chunk, chunk)], buf_b, sem_b)
            ra.wait()
            sa = [pltpu.async_copy(buf_a, out_hbm.at[idx_v.at[kk].at[c]],
                                   sem_s) for kk in range(K)]
            rb.wait()
            sb = [pltpu.async_copy(buf_b, out_hbm.at[idx_v.at[kk].at[c + 1]],
                                   sem_s) for kk in range(K)]
            for cp in sa + sb:
                cp.wait()

    return k(x, idx3)


def _sc_unpermute(y, pos_kt, chunk):
    H = y.shape[1]
    n = pos_kt.shape[0]
    rows_per_w = n // NUM_WORKERS
    nchunks = rows_per_w // chunk
    idx3 = pos_kt.reshape(NUM_WORKERS, nchunks, chunk)
    mesh = plsc.VectorSubcoreMesh(core_axis_name="c", subcore_axis_name="s")

    @functools.partial(
        pl.kernel,
        mesh=mesh,
        out_type=jax.ShapeDtypeStruct((n, H), y.dtype),
        scratch_types=[
            pltpu.VMEM((nchunks, chunk), jnp.int32),
            pltpu.VMEM((chunk, H), y.dtype),
            pltpu.VMEM((chunk, H), y.dtype),
            pltpu.SemaphoreType.DMA,
            pltpu.SemaphoreType.DMA,
            pltpu.SemaphoreType.DMA,
        ],
    )
    def k(y_hbm, idx_hbm, out_hbm, idx_v, buf_a, buf_b, sem_a, sem_b, sem_s):
        wid = lax.axis_index("s") * 2 + lax.axis_index("c")
        pltpu.async_copy(idx_hbm.at[wid], idx_v, sem_a).wait()
        base = wid * rows_per_w

        @pl.loop(0, nchunks, step=2)
        def _(c):
            ga = pltpu.async_copy(y_hbm.at[idx_v.at[c]], buf_a, sem_a)
            gb = pltpu.async_copy(y_hbm.at[idx_v.at[c + 1]], buf_b, sem_b)
            ga.wait()
            wa = pltpu.async_copy(
                buf_a, out_hbm.at[pl.ds(base + c * chunk, chunk)], sem_s)
            gb.wait()
            wb = pltpu.async_copy(
                buf_b, out_hbm.at[pl.ds(base + (c + 1) * chunk, chunk)], sem_s)
            wa.wait()
            wb.wait()

    return k(y, idx3)


def _ffn_body(be_ref, bv_ref, xs_ref, w1_ref, w3_ref, w2_ref, y_ref):
    @pl.when(bv_ref[pl.program_id(0)] != 0)
    def _():
        xb = xs_ref[...].astype(jnp.bfloat16)
        w1b = w1_ref[0].astype(jnp.bfloat16)
        w3b = w3_ref[0].astype(jnp.bfloat16)
        w2b = w2_ref[0].astype(jnp.bfloat16)
        g = jnp.dot(xb, w1b, preferred_element_type=jnp.float32)
        u = jnp.dot(xb, w3b, preferred_element_type=jnp.float32)
        h = ((g * lax.logistic(g)) * u).astype(jnp.bfloat16)
        y_ref[...] = jnp.dot(h, w2b, preferred_element_type=jnp.float32)


def _grouped_ffn(block_expert, block_valid, xs, w1, w3, w2, n_blocks,
                 interpret=False):
    npad, H = xs.shape
    I = w1.shape[2]
    grid_spec = pltpu.PrefetchScalarGridSpec(
        num_scalar_prefetch=2,
        grid=(n_blocks,),
        in_specs=[
            pl.BlockSpec((BM, H), lambda i, be, bv: (i, 0)),
            pl.BlockSpec((1, H, I), lambda i, be, bv: (be[i], 0, 0)),
            pl.BlockSpec((1, H, I), lambda i, be, bv: (be[i], 0, 0)),
            pl.BlockSpec((1, I, H), lambda i, be, bv: (be[i], 0, 0)),
        ],
        out_specs=pl.BlockSpec((BM, H), lambda i, be, bv: (i, 0)),
    )
    return pl.pallas_call(
        _ffn_body,
        grid_spec=grid_spec,
        out_shape=jax.ShapeDtypeStruct((npad, H), jnp.float32),
        compiler_params=pltpu.CompilerParams(
            vmem_limit_bytes=64 * 1024 * 1024),
        interpret=interpret,
    )(block_expert, block_valid, xs, w1, w3, w2)


def _combine_body(*refs):
    out_ref = refs[-1]
    acc = refs[0][...] * refs[1][...]
    for j in range(2, len(refs) - 1, 2):
        acc = acc + refs[j][...] * refs[j + 1][...]
    out_ref[...] = acc


def _combine(yu, expert_weights, T, K, interpret=False):
    H = yu.shape[1]
    BT = 512
    nb = T // BT
    ew_col = expert_weights.T.reshape(K * T, 1)
    in_specs = []
    args = []
    for k in range(K):
        in_specs.append(
            pl.BlockSpec((BT, H),
                         functools.partial(lambda k, i: (k * nb + i, 0), k)))
        args.append(yu)
        in_specs.append(
            pl.BlockSpec((BT, 1),
                         functools.partial(lambda k, i: (k * nb + i, 0), k)))
        args.append(ew_col)
    return pl.pallas_call(
        _combine_body,
        grid=(nb,),
        in_specs=in_specs,
        out_specs=pl.BlockSpec((BT, H), lambda i: (i, 0)),
        out_shape=jax.ShapeDtypeStruct((T, H), jnp.float32),
        interpret=interpret,
    )(*args)


def kernel(x, expert_indices, expert_weights, w1, w2, w3):
    T, H = x.shape
    K = expert_indices.shape[1]
    E = w1.shape[0]

    block_expert, block_valid, pos_k, pos_kt, n_blocks = _routing_metadata(
        expert_indices, E)
    npad = n_blocks * BM

    xs = _sc_permute(x, pos_k, npad, chunk=32)
    y = _grouped_ffn(block_expert, block_valid, xs, w1, w3, w2, n_blocks)
    yu = _sc_unpermute(y, pos_kt, chunk=32)
    return _combine(yu, expert_weights.astype(jnp.float32), T, K)
